# baseline (speedup 1.0000x reference)
import functools

import jax
import jax.numpy as jnp
from jax import lax
from jax.experimental import pallas as pl
from jax.experimental.pallas import tpu as pltpu
from jax.experimental.pallas import tpu_sc as plsc

DIM = 64
NENT = 1000000
NREL = 1000
B = 16384
NC, NS, L = 2, 16, 16
NW = NC * NS
PER_W = B // NW
CHUNK = 128
NCHUNK = PER_W // CHUNK
GROUPS = CHUNK // L


def _build_score():
    mesh = plsc.VectorSubcoreMesh(core_axis_name="c", subcore_axis_name="s")
    out_t = (jax.ShapeDtypeStruct((B,), jnp.float32),
             jax.ShapeDtypeStruct((B,), jnp.float32))
    scratch = [
        pltpu.VMEM((PER_W,), jnp.int32),
        pltpu.VMEM((PER_W,), jnp.int32),
        pltpu.VMEM((PER_W,), jnp.int32),
        pltpu.VMEM((PER_W,), jnp.int32),
        pltpu.VMEM((PER_W,), jnp.int32),
        pltpu.VMEM((PER_W,), jnp.int32),
        pltpu.VMEM((CHUNK, 2 * DIM), jnp.float32),
        pltpu.VMEM((CHUNK, 2 * DIM), jnp.float32),
        pltpu.VMEM((CHUNK, 2 * DIM), jnp.float32),
        pltpu.VMEM((PER_W,), jnp.float32),
        pltpu.SemaphoreType.DMA,
        pltpu.SemaphoreType.DMA,
        pltpu.SemaphoreType.DMA,
    ]

    @functools.partial(
        pl.kernel, out_type=out_t, mesh=mesh, scratch_types=scratch,
        compiler_params=pltpu.CompilerParams(needs_layout_passes=False,
                                             use_tc_tiling_on_sc=True))
    def trans_e(pos_h, pos_r, pos_t, neg_h, neg_r, neg_t, ent2, rel2,
                pos_out, neg_out,
                hidx, ridx, tidx, hoff, roff, toff, hrows, rrows, trows,
                scores, sem_h, sem_r, sem_t):
        wid = lax.axis_index("s") * NC + lax.axis_index("c")
        base = wid * PER_W
        lanes = lax.iota(jnp.int32, L)
        for h_in, r_in, t_in, out in ((pos_h, pos_r, pos_t, pos_out),
                                      (neg_h, neg_r, neg_t, neg_out)):
            pltpu.sync_copy(h_in.at[pl.ds(base, PER_W)], hidx)
            pltpu.sync_copy(r_in.at[pl.ds(base, PER_W)], ridx)
            pltpu.sync_copy(t_in.at[pl.ds(base, PER_W)], tidx)

            @pl.loop(0, PER_W // L)
            def _split(g):
                for idx, off in ((hidx, hoff), (ridx, roff), (tidx, toff)):
                    v = idx[pl.ds(g * L, L)]
                    off[pl.ds(g * L, L)] = (v & 1) * DIM
                    idx[pl.ds(g * L, L)] = v >> 1

            @pl.loop(0, NCHUNK)
            def _chunk(c):
                off = c * CHUNK
                cp_h = pltpu.async_copy(
                    ent2.at[hidx.at[pl.ds(off, CHUNK)]], hrows, sem_h)
                cp_r = pltpu.async_copy(
                    rel2.at[ridx.at[pl.ds(off, CHUNK)]], rrows, sem_r)
                cp_t = pltpu.async_copy(
                    ent2.at[tidx.at[pl.ds(off, CHUNK)]], trows, sem_t)
                cp_h.wait()
                cp_r.wait()
                cp_t.wait()

                @pl.loop(0, GROUPS)
                def _group(g):
                    rowv = g * L + lanes
                    hc = hoff[pl.ds(off + g * L, L)]
                    rc = roff[pl.ds(off + g * L, L)]
                    tc = toff[pl.ds(off + g * L, L)]
                    acc = jnp.zeros((L,), jnp.float32)
                    for d in range(DIM):
                        hv = plsc.load_gather(hrows, [rowv, hc + d])
                        rv = plsc.load_gather(rrows, [rowv, rc + d])
                        tv = plsc.load_gather(trows, [rowv, tc + d])
                        acc = acc + jnp.abs(hv + rv - tv)
                    scores[pl.ds(off + g * L, L)] = acc

            pltpu.sync_copy(scores, out.at[pl.ds(base, PER_W)])

    return trans_e


_score = _build_score()


def kernel(pos_h, pos_r, pos_t, neg_h, neg_r, neg_t, entity_emb, relation_emb):
    ent2 = entity_emb.reshape(NENT // 2, 2 * DIM)
    rel2 = relation_emb.reshape(NREL // 2, 2 * DIM)
    return _score(pos_h, pos_r, pos_t, neg_h, neg_r, neg_t, ent2, rel2)
